# baseline (speedup 1.0000x reference)
import functools

import jax
import jax.numpy as jnp
from jax import lax
from jax.experimental import pallas as pl
from jax.experimental.pallas import tpu as pltpu
from jax.experimental.pallas import tpu_sc as plsc

N = 10000
E = 160000
F = 256
H = 256
O = 128

NC = 2
NS = 16
NW = NC * NS
CHUNK = 128
E_PAD = 196608
NCHUNKS = E_PAD // CHUNK
CPW = NCHUNKS // NW
CPT = NCHUNKS // NS
NP = 10240
ROWS_PER_TILE = NP // NS

_mesh = plsc.VectorSubcoreMesh(core_axis_name="c", subcore_axis_name="s")
_sc_params = pltpu.CompilerParams(needs_layout_passes=False,
                                  use_tc_tiling_on_sc=False)


def _zero_rows(buf, nrows, width):
    def body(i, _):
        for j in range(width // 16):
            buf[i, pl.ds(16 * j, 16)] = jnp.zeros((16,), jnp.float32)
        return 0
    lax.fori_loop(0, nrows, body, 0)


def _deg_body(dst_hbm, ew_hbm, out_hbm, dstb, ewb, vbuf, zbuf, acc, sem):
    cid = lax.axis_index("c")
    sid = lax.axis_index("s")
    w = sid * NC + cid

    _zero_rows(zbuf, ROWS_PER_TILE, 16)
    pltpu.sync_copy(zbuf, acc.at[pl.ds(sid * ROWS_PER_TILE, ROWS_PER_TILE)])
    plsc.subcore_barrier()

    pltpu.sync_copy(dst_hbm.at[pl.ds(w * CPW, CPW)], dstb)
    pltpu.sync_copy(ew_hbm.at[pl.ds(w * CPW, CPW)], ewb)

    def chunk_body(c, _):
        def fill(g, _):
            wv = ewb[c, pl.ds(16 * g, 16)]
            for l in range(16):
                vbuf[16 * g + l, :] = jnp.full((16,), wv[l], jnp.float32)
            return 0
        lax.fori_loop(0, CHUNK // 16, fill, 0)
        pltpu.sync_copy(vbuf, acc.at[dstb.at[c]], add=True)
        return 0

    lax.fori_loop(0, CPW, chunk_body, 0)
    plsc.subcore_barrier()

    pltpu.sync_copy(acc.at[pl.ds(sid * ROWS_PER_TILE, ROWS_PER_TILE)], zbuf)
    pltpu.sync_copy(zbuf, out_hbm.at[cid, pl.ds(sid * ROWS_PER_TILE, ROWS_PER_TILE)])


_deg_call = pl.kernel(
    _deg_body,
    out_type=jax.ShapeDtypeStruct((NC, NP, 16), jnp.float32),
    mesh=_mesh,
    scratch_types=[
        pltpu.VMEM((CPW, CHUNK), jnp.int32),
        pltpu.VMEM((CPW, CHUNK), jnp.float32),
        pltpu.VMEM((CHUNK, 16), jnp.float32),
        pltpu.VMEM((ROWS_PER_TILE, 16), jnp.float32),
        pltpu.VMEM_SHARED((NP, 16), jnp.float32),
        pltpu.SemaphoreType.DMA,
    ],
    compiler_params=_sc_params,
)


def _dinv_tc(deg_ref, o_ref):
    d = deg_ref[0] + deg_ref[1]
    o_ref[...] = lax.rsqrt(d)


def _dinv_call(degacc):
    return pl.pallas_call(
        _dinv_tc,
        out_shape=jax.ShapeDtypeStruct((NP, 16), jnp.float32),
    )(degacc)


def _norm_body(src_hbm, dst_hbm, ew_hbm, dinv_hbm, out_hbm,
               srcb, dstb, ewb, normb, dinvb, sem):
    cid = lax.axis_index("c")
    sid = lax.axis_index("s")
    w = sid * NC + cid

    pltpu.sync_copy(dinv_hbm, dinvb)
    pltpu.sync_copy(src_hbm.at[pl.ds(w * CPW, CPW)], srcb)
    pltpu.sync_copy(dst_hbm.at[pl.ds(w * CPW, CPW)], dstb)
    pltpu.sync_copy(ew_hbm.at[pl.ds(w * CPW, CPW)], ewb)

    def chunk_body(c, _):
        for g in range(CHUNK // 16):
            sl = pl.ds(16 * g, 16)
            isrc = srcb[c, sl]
            idst = dstb[c, sl]
            ds_ = plsc.load_gather(dinvb, [isrc])
            dd = plsc.load_gather(dinvb, [idst])
            normb[c, sl] = ds_ * ewb[c, sl] * dd
        return 0

    lax.fori_loop(0, CPW, chunk_body, 0)
    pltpu.sync_copy(normb, out_hbm.at[pl.ds(w * CPW, CPW)])


_norm_call = pl.kernel(
    _norm_body,
    out_type=jax.ShapeDtypeStruct((NCHUNKS, CHUNK), jnp.float32),
    mesh=_mesh,
    scratch_types=[
        pltpu.VMEM((CPW, CHUNK), jnp.int32),
        pltpu.VMEM((CPW, CHUNK), jnp.int32),
        pltpu.VMEM((CPW, CHUNK), jnp.float32),
        pltpu.VMEM((CPW, CHUNK), jnp.float32),
        pltpu.VMEM((NP,), jnp.float32),
        pltpu.SemaphoreType.DMA,
    ],
    compiler_params=_sc_params,
)


SPH = 24


def _make_agg(col_split, cpt):
    D = 128
    phases = cpt // SPH

    def body(src_hbm, dst_hbm, norm_hbm, rows_hbm, out_hbm,
             srcb, dstb, normb, rows0, rows1, acc,
             semg0, semg1, sems0, sems1):
        cid = lax.axis_index("c")
        sid = lax.axis_index("s")
        if col_split:
            tbase = sid * cpt
        else:
            tbase = (sid * NC + cid) * cpt
        rows = (rows0, rows1)
        semg = (semg0, semg1)
        sems = (sems0, sems1)

        _zero_rows(rows0, CHUNK, D)
        for r in range(ROWS_PER_TILE // CHUNK):
            pltpu.sync_copy(
                rows0, acc.at[pl.ds(sid * ROWS_PER_TILE + r * CHUNK, CHUNK)])
        plsc.subcore_barrier()

        def scale_rows(buf, c):
            def scale(g, _):
                nv = normb[c, pl.ds(16 * g, 16)]
                for l in range(16):
                    sv = jnp.take(nv, jnp.full((16,), l, jnp.int32))
                    e = 16 * g + l
                    for j in range(D // 16):
                        sl = pl.ds(16 * j, 16)
                        buf[e, sl] = buf[e, sl] * sv
                return 0
            lax.fori_loop(0, CHUNK // 16, scale, 0)

        for ph in range(phases):
            base = tbase + ph * SPH
            pltpu.sync_copy(src_hbm.at[pl.ds(base, SPH)], srcb)
            pltpu.sync_copy(dst_hbm.at[pl.ds(base, SPH)], dstb)
            pltpu.sync_copy(norm_hbm.at[pl.ds(base, SPH)], normb)

            if col_split:
                off = (cid * NP).astype(jnp.int32)

                def adjust(c, _):
                    for g in range(CHUNK // 16):
                        sl = pl.ds(16 * g, 16)
                        srcb[c, sl] = srcb[c, sl] + off
                    return 0
                lax.fori_loop(0, SPH, adjust, 0)

            pltpu.async_copy(rows_hbm.at[srcb.at[0]], rows0, semg0)

            def pair_body(k, _):
                for b in range(2):
                    c = 2 * k + b
                    nb = 1 - b

                    @pl.when(c + 1 < SPH)
                    def _():
                        pltpu.async_copy(
                            rows_hbm.at[srcb.at[c + 1]], rows[nb], semg[nb])

                    pltpu.make_async_copy(
                        rows_hbm.at[srcb.at[c]], rows[b], semg[b]).wait()
                    scale_rows(rows[b], c)
                    pltpu.sync_copy(rows[b], acc.at[dstb.at[c]], add=True)
                return 0

            lax.fori_loop(0, SPH // 2, pair_body, 0)

        plsc.subcore_barrier()

        for r in range(ROWS_PER_TILE // CHUNK):
            rbase = sid * ROWS_PER_TILE + r * CHUNK
            pltpu.sync_copy(acc.at[pl.ds(rbase, CHUNK)], rows0)
            pltpu.sync_copy(rows0, out_hbm.at[cid, pl.ds(rbase, CHUNK)])

    return pl.kernel(
        body,
        out_type=jax.ShapeDtypeStruct((NC, NP, D), jnp.float32),
        mesh=_mesh,
        scratch_types=[
            pltpu.VMEM((SPH, CHUNK), jnp.int32),
            pltpu.VMEM((SPH, CHUNK), jnp.int32),
            pltpu.VMEM((SPH, CHUNK), jnp.float32),
            pltpu.VMEM((CHUNK, D), jnp.float32),
            pltpu.VMEM((CHUNK, D), jnp.float32),
            pltpu.VMEM_SHARED((NP, D), jnp.float32),
            pltpu.SemaphoreType.DMA,
            pltpu.SemaphoreType.DMA,
            pltpu.SemaphoreType.DMA,
            pltpu.SemaphoreType.DMA,
        ],
        compiler_params=_sc_params,
    )


_agg_l1 = _make_agg(col_split=True, cpt=CPT)
_agg_l2 = _make_agg(col_split=False, cpt=CPW)


_RB = 1024


def _mm1_tc(x_ref, w_ref, o_ref):
    o_ref[0] = jnp.dot(x_ref[...], w_ref[...],
                       preferred_element_type=jnp.float32)


def _mm1_call(x, W1):
    return pl.pallas_call(
        _mm1_tc,
        out_shape=jax.ShapeDtypeStruct((NC, NP, 128), jnp.float32),
        grid=(NP // _RB, NC),
        in_specs=[
            pl.BlockSpec((_RB, F), lambda i, j: (i, 0)),
            pl.BlockSpec((F, 128), lambda i, j: (0, j)),
        ],
        out_specs=pl.BlockSpec((1, _RB, 128), lambda i, j: (j, i, 0)),
    )(x, W1)


def _mm2_tc(a_ref, b1_ref, w2a_ref, w2b_ref, o_ref):
    hlo = jax.nn.relu(a_ref[0] + b1_ref[0:1, :])
    hhi = jax.nn.relu(a_ref[1] + b1_ref[1:2, :])
    o_ref[...] = (
        jnp.dot(hlo, w2a_ref[...], preferred_element_type=jnp.float32)
        + jnp.dot(hhi, w2b_ref[...], preferred_element_type=jnp.float32)
    )


def _mm2_call(agg1, b1r, W2a, W2b):
    return pl.pallas_call(
        _mm2_tc,
        out_shape=jax.ShapeDtypeStruct((NP, O), jnp.float32),
        grid=(NP // _RB,),
        in_specs=[
            pl.BlockSpec((2, _RB, 128), lambda i: (0, i, 0)),
            pl.BlockSpec((2, 128), lambda i: (0, 0)),
            pl.BlockSpec((128, O), lambda i: (0, 0)),
            pl.BlockSpec((128, O), lambda i: (0, 0)),
        ],
        out_specs=pl.BlockSpec((_RB, O), lambda i: (i, 0)),
    )(agg1, b1r, W2a, W2b)


def _final_tc(p_ref, b2_ref, o_ref):
    o_ref[...] = jax.nn.sigmoid(p_ref[0] + p_ref[1] + b2_ref[...])


def _final_call(parts, b2r):
    return pl.pallas_call(
        _final_tc,
        out_shape=jax.ShapeDtypeStruct((NP, O), jnp.float32),
        grid=(NP // _RB,),
        in_specs=[
            pl.BlockSpec((2, _RB, O), lambda i: (0, i, 0)),
            pl.BlockSpec((1, O), lambda i: (0, 0)),
        ],
        out_specs=pl.BlockSpec((_RB, O), lambda i: (i, 0)),
    )(parts, b2r)


def kernel(x, ei, ew, W1, b1, W2, b2):
    src = ei[0].astype(jnp.int32)
    dst = ei[1].astype(jnp.int32)
    loop = jnp.arange(N, dtype=jnp.int32)
    pad = E_PAD - (E + N)
    zi = jnp.zeros((pad,), jnp.int32)
    src_f = jnp.concatenate([src, loop, zi]).reshape(NCHUNKS, CHUNK)
    dst_f = jnp.concatenate([dst, loop, zi]).reshape(NCHUNKS, CHUNK)
    ew_f = jnp.concatenate(
        [ew, jnp.ones((N,), jnp.float32), jnp.zeros((pad,), jnp.float32)]
    ).reshape(NCHUNKS, CHUNK)

    degacc = _deg_call(dst_f, ew_f)
    dinv16 = _dinv_call(degacc)
    dinv = dinv16[:, 0]
    norm = _norm_call(src_f, dst_f, ew_f, dinv)

    xp = jnp.pad(x, ((0, NP - N), (0, 0)))
    xw2 = _mm1_call(xp, W1).reshape(NC * NP, 128)
    agg1 = _agg_l1(src_f, dst_f, norm, xw2)

    b1r = b1.reshape(2, 128)
    W2a = W2[:128]
    W2b = W2[128:]
    hw2 = _mm2_call(agg1, b1r, W2a, W2b)

    parts = _agg_l2(src_f, dst_f, norm, hw2)
    out = _final_call(parts, b2.reshape(1, O))
    return out[:N]


# speedup vs baseline: 1.6983x; 1.6983x over previous
import functools

import jax
import jax.numpy as jnp
from jax import lax
from jax.experimental import pallas as pl
from jax.experimental.pallas import tpu as pltpu
from jax.experimental.pallas import tpu_sc as plsc

N = 10000
E = 160000
F = 256
H = 256
O = 128

NC = 2
NS = 16
NW = NC * NS
CHUNK = 128
E_PAD = 196608
NCHUNKS = E_PAD // CHUNK
CPW = NCHUNKS // NW
CPT = NCHUNKS // NS
NP = 10240
ROWS_PER_TILE = NP // NS

_mesh = plsc.VectorSubcoreMesh(core_axis_name="c", subcore_axis_name="s")
_sc_params = pltpu.CompilerParams(needs_layout_passes=False,
                                  use_tc_tiling_on_sc=False)


def _zero_rows(buf, nrows, width):
    def body(i, _):
        for j in range(width // 16):
            buf[i, pl.ds(16 * j, 16)] = jnp.zeros((16,), jnp.float32)
        return 0
    lax.fori_loop(0, nrows, body, 0)


def _deg_body(dst_hbm, ew_hbm, out_hbm, dstb, ewb, vbuf, zbuf, acc, sem):
    cid = lax.axis_index("c")
    sid = lax.axis_index("s")
    w = sid * NC + cid

    _zero_rows(zbuf, ROWS_PER_TILE, 16)
    pltpu.sync_copy(zbuf, acc.at[pl.ds(sid * ROWS_PER_TILE, ROWS_PER_TILE)])
    plsc.subcore_barrier()

    pltpu.sync_copy(dst_hbm.at[pl.ds(w * CPW, CPW)], dstb)
    pltpu.sync_copy(ew_hbm.at[pl.ds(w * CPW, CPW)], ewb)

    def chunk_body(c, _):
        def fill(g, _):
            wv = ewb[c, pl.ds(16 * g, 16)]
            for l in range(16):
                vbuf[16 * g + l, :] = jnp.full((16,), wv[l], jnp.float32)
            return 0
        lax.fori_loop(0, CHUNK // 16, fill, 0)
        pltpu.sync_copy(vbuf, acc.at[dstb.at[c]], add=True)
        return 0

    lax.fori_loop(0, CPW, chunk_body, 0)
    plsc.subcore_barrier()

    pltpu.sync_copy(acc.at[pl.ds(sid * ROWS_PER_TILE, ROWS_PER_TILE)], zbuf)
    pltpu.sync_copy(zbuf, out_hbm.at[cid, pl.ds(sid * ROWS_PER_TILE, ROWS_PER_TILE)])


_deg_call = pl.kernel(
    _deg_body,
    out_type=jax.ShapeDtypeStruct((NC, NP, 16), jnp.float32),
    mesh=_mesh,
    scratch_types=[
        pltpu.VMEM((CPW, CHUNK), jnp.int32),
        pltpu.VMEM((CPW, CHUNK), jnp.float32),
        pltpu.VMEM((CHUNK, 16), jnp.float32),
        pltpu.VMEM((ROWS_PER_TILE, 16), jnp.float32),
        pltpu.VMEM_SHARED((NP, 16), jnp.float32),
        pltpu.SemaphoreType.DMA,
    ],
    compiler_params=_sc_params,
)


def _dinv_tc(deg_ref, o_ref):
    d = deg_ref[0] + deg_ref[1]
    o_ref[...] = lax.rsqrt(d)


def _dinv_call(degacc):
    return pl.pallas_call(
        _dinv_tc,
        out_shape=jax.ShapeDtypeStruct((NP, 16), jnp.float32),
    )(degacc)


def _norm_body(src_hbm, dst_hbm, ew_hbm, dinv_hbm, out_hbm,
               srcb, dstb, ewb, normb, dinvb, sem):
    cid = lax.axis_index("c")
    sid = lax.axis_index("s")
    w = sid * NC + cid

    pltpu.sync_copy(dinv_hbm, dinvb)
    pltpu.sync_copy(src_hbm.at[pl.ds(w * CPW, CPW)], srcb)
    pltpu.sync_copy(dst_hbm.at[pl.ds(w * CPW, CPW)], dstb)
    pltpu.sync_copy(ew_hbm.at[pl.ds(w * CPW, CPW)], ewb)

    def chunk_body(c, _):
        for g in range(CHUNK // 16):
            sl = pl.ds(16 * g, 16)
            isrc = srcb[c, sl]
            idst = dstb[c, sl]
            ds_ = plsc.load_gather(dinvb, [isrc])
            dd = plsc.load_gather(dinvb, [idst])
            normb[c, sl] = ds_ * ewb[c, sl] * dd
        return 0

    lax.fori_loop(0, CPW, chunk_body, 0)
    pltpu.sync_copy(normb, out_hbm.at[pl.ds(w * CPW, CPW)])


_norm_call = pl.kernel(
    _norm_body,
    out_type=jax.ShapeDtypeStruct((NCHUNKS, CHUNK), jnp.float32),
    mesh=_mesh,
    scratch_types=[
        pltpu.VMEM((CPW, CHUNK), jnp.int32),
        pltpu.VMEM((CPW, CHUNK), jnp.int32),
        pltpu.VMEM((CPW, CHUNK), jnp.float32),
        pltpu.VMEM((CPW, CHUNK), jnp.float32),
        pltpu.VMEM((NP,), jnp.float32),
        pltpu.SemaphoreType.DMA,
    ],
    compiler_params=_sc_params,
)


SPH = 24


def _make_agg(col_split, cpt):
    D = 128
    phases = cpt // SPH

    def body(src_hbm, dst_hbm, norm_hbm, rows_hbm, out_hbm,
             srcb, dstb, normb, rbf0, rbf1, rows_f, acc, semg0, semg1):
        cid = lax.axis_index("c")
        sid = lax.axis_index("s")
        if col_split:
            tbase = sid * cpt
        else:
            tbase = (sid * NC + cid) * cpt
        rbf = (rbf0, rbf1)
        semg = (semg0, semg1)

        _zero_rows(rows_f, CHUNK, D)
        for r in range(ROWS_PER_TILE // CHUNK):
            pltpu.sync_copy(
                rows_f, acc.at[pl.ds(sid * ROWS_PER_TILE + r * CHUNK, CHUNK)])
        plsc.subcore_barrier()

        def scale_rows(buf, c):
            def scale(g, _):
                nv = normb[c, pl.ds(16 * g, 16)]
                for l in range(16):
                    sv = jnp.take(nv, jnp.full((16,), l, jnp.int32))
                    e = 16 * g + l
                    for j in range(4):
                        v = buf[e, pl.ds(32 * j, 32)]
                        pa, pb = plsc.unpack(
                            v, format=plsc.PackFormat.INTERLEAVED)
                        rows_f[e, pl.ds(16 * j, 16)] = pa * sv
                        rows_f[e, pl.ds(64 + 16 * j, 16)] = pb * sv
                return 0
            lax.fori_loop(0, CHUNK // 16, scale, 0)

        for ph in range(phases):
            base = tbase + ph * SPH
            pltpu.sync_copy(src_hbm.at[pl.ds(base, SPH)], srcb)
            pltpu.sync_copy(dst_hbm.at[pl.ds(base, SPH)], dstb)
            pltpu.sync_copy(norm_hbm.at[pl.ds(base, SPH)], normb)

            if col_split:
                off = (cid * NP).astype(jnp.int32)

                def adjust(c, _):
                    for g in range(CHUNK // 16):
                        sl = pl.ds(16 * g, 16)
                        srcb[c, sl] = srcb[c, sl] + off
                    return 0
                lax.fori_loop(0, SPH, adjust, 0)

            pltpu.async_copy(rows_hbm.at[srcb.at[0]], rbf0, semg0)

            def pair_body(k, _):
                for b in range(2):
                    c = 2 * k + b
                    nb = 1 - b

                    @pl.when(c + 1 < SPH)
                    def _():
                        pltpu.async_copy(
                            rows_hbm.at[srcb.at[c + 1]], rbf[nb], semg[nb])

                    pltpu.make_async_copy(
                        rows_hbm.at[srcb.at[c]], rbf[b], semg[b]).wait()
                    scale_rows(rbf[b], c)
                    pltpu.sync_copy(rows_f, acc.at[dstb.at[c]], add=True)
                return 0

            lax.fori_loop(0, SPH // 2, pair_body, 0)

        plsc.subcore_barrier()

        for r in range(ROWS_PER_TILE // CHUNK):
            rbase = sid * ROWS_PER_TILE + r * CHUNK
            pltpu.sync_copy(acc.at[pl.ds(rbase, CHUNK)], rows_f)
            pltpu.sync_copy(rows_f, out_hbm.at[cid, pl.ds(rbase, CHUNK)])

    return pl.kernel(
        body,
        out_type=jax.ShapeDtypeStruct((NC, NP, D), jnp.float32),
        mesh=_mesh,
        scratch_types=[
            pltpu.VMEM((SPH, CHUNK), jnp.int32),
            pltpu.VMEM((SPH, CHUNK), jnp.int32),
            pltpu.VMEM((SPH, CHUNK), jnp.float32),
            pltpu.VMEM((CHUNK, D), jnp.bfloat16),
            pltpu.VMEM((CHUNK, D), jnp.bfloat16),
            pltpu.VMEM((CHUNK, D), jnp.float32),
            pltpu.VMEM_SHARED((NP, D), jnp.float32),
            pltpu.SemaphoreType.DMA,
            pltpu.SemaphoreType.DMA,
        ],
        compiler_params=_sc_params,
    )


_agg_l1 = _make_agg(col_split=True, cpt=CPT)
_agg_l2 = _make_agg(col_split=False, cpt=CPW)


_RB = 1024


def _mm1_tc(x_ref, w_ref, o_ref):
    o_ref[0] = jnp.dot(x_ref[...], w_ref[...],
                       preferred_element_type=jnp.float32
                       ).astype(jnp.bfloat16)


def _mm1_call(x, W1):
    return pl.pallas_call(
        _mm1_tc,
        out_shape=jax.ShapeDtypeStruct((NC, NP, 128), jnp.bfloat16),
        grid=(NP // _RB, NC),
        in_specs=[
            pl.BlockSpec((_RB, F), lambda i, j: (i, 0)),
            pl.BlockSpec((F, 128), lambda i, j: (0, j)),
        ],
        out_specs=pl.BlockSpec((1, _RB, 128), lambda i, j: (j, i, 0)),
    )(x, W1)


def _mm2_tc(a_ref, b1_ref, w2a_ref, w2b_ref, o_ref):
    hlo = jax.nn.relu(a_ref[0] + b1_ref[0:1, :])
    hhi = jax.nn.relu(a_ref[1] + b1_ref[1:2, :])
    o_ref[...] = (
        jnp.dot(hlo, w2a_ref[...], preferred_element_type=jnp.float32)
        + jnp.dot(hhi, w2b_ref[...], preferred_element_type=jnp.float32)
    ).astype(jnp.bfloat16)


def _mm2_call(agg1, b1r, W2a, W2b):
    return pl.pallas_call(
        _mm2_tc,
        out_shape=jax.ShapeDtypeStruct((NP, O), jnp.bfloat16),
        grid=(NP // _RB,),
        in_specs=[
            pl.BlockSpec((2, _RB, 128), lambda i: (0, i, 0)),
            pl.BlockSpec((2, 128), lambda i: (0, 0)),
            pl.BlockSpec((128, O), lambda i: (0, 0)),
            pl.BlockSpec((128, O), lambda i: (0, 0)),
        ],
        out_specs=pl.BlockSpec((_RB, O), lambda i: (i, 0)),
    )(agg1, b1r, W2a, W2b)


def _final_tc(p_ref, b2_ref, o_ref):
    o_ref[...] = jax.nn.sigmoid(p_ref[0] + p_ref[1] + b2_ref[...])


def _final_call(parts, b2r):
    return pl.pallas_call(
        _final_tc,
        out_shape=jax.ShapeDtypeStruct((NP, O), jnp.float32),
        grid=(NP // _RB,),
        in_specs=[
            pl.BlockSpec((2, _RB, O), lambda i: (0, i, 0)),
            pl.BlockSpec((1, O), lambda i: (0, 0)),
        ],
        out_specs=pl.BlockSpec((_RB, O), lambda i: (i, 0)),
    )(parts, b2r)


def kernel(x, ei, ew, W1, b1, W2, b2):
    src = ei[0].astype(jnp.int32)
    dst = ei[1].astype(jnp.int32)
    loop = jnp.arange(N, dtype=jnp.int32)
    pad = E_PAD - (E + N)
    zi = jnp.zeros((pad,), jnp.int32)
    src_f = jnp.concatenate([src, loop, zi]).reshape(NCHUNKS, CHUNK)
    dst_f = jnp.concatenate([dst, loop, zi]).reshape(NCHUNKS, CHUNK)
    ew_f = jnp.concatenate(
        [ew, jnp.ones((N,), jnp.float32), jnp.zeros((pad,), jnp.float32)]
    ).reshape(NCHUNKS, CHUNK)

    degacc = _deg_call(dst_f, ew_f)
    dinv16 = _dinv_call(degacc)
    dinv = dinv16[:, 0]
    norm = _norm_call(src_f, dst_f, ew_f, dinv)

    xp = jnp.pad(x, ((0, NP - N), (0, 0)))
    pA = jnp.arange(64).reshape(4, 16)
    p128 = jnp.stack([pA, pA + 64], axis=-1).reshape(-1)
    W1p = jnp.concatenate([W1[:, :128][:, p128], W1[:, 128:][:, p128]], axis=1)
    xw2 = _mm1_call(xp, W1p).reshape(NC * NP, 128)
    agg1 = _agg_l1(src_f, dst_f, norm, xw2)

    b1r = b1.reshape(2, 128)
    W2a = W2[:128][:, p128]
    W2b = W2[128:][:, p128]
    hw2 = _mm2_call(agg1, b1r, W2a, W2b)

    parts = _agg_l2(src_f, dst_f, norm, hw2)
    out = _final_call(parts, b2.reshape(1, O))
    return out[:N]


# speedup vs baseline: 1.7610x; 1.0369x over previous
import functools

import jax
import jax.numpy as jnp
from jax import lax
from jax.experimental import pallas as pl
from jax.experimental.pallas import tpu as pltpu
from jax.experimental.pallas import tpu_sc as plsc

N = 10000
E = 160000
F = 256
H = 256
O = 128

NC = 2
NS = 16
NW = NC * NS
CHUNK = 128
E_PAD = 196608
NCHUNKS = E_PAD // CHUNK
CPW = NCHUNKS // NW
CPT = NCHUNKS // NS
NP = 10240
ROWS_PER_TILE = NP // NS

_mesh = plsc.VectorSubcoreMesh(core_axis_name="c", subcore_axis_name="s")
_sc_params = pltpu.CompilerParams(needs_layout_passes=False,
                                  use_tc_tiling_on_sc=False)


def _zero_rows(buf, nrows, width):
    def body(i, _):
        for j in range(width // 16):
            buf[i, pl.ds(16 * j, 16)] = jnp.zeros((16,), jnp.float32)
        return 0
    lax.fori_loop(0, nrows, body, 0)


def _deg_body(dst_hbm, ew_hbm, out_hbm, dstb, ewb, vbuf, zbuf, acc, sem):
    cid = lax.axis_index("c")
    sid = lax.axis_index("s")
    w = sid * NC + cid

    _zero_rows(zbuf, ROWS_PER_TILE, 16)
    pltpu.sync_copy(zbuf, acc.at[pl.ds(sid * ROWS_PER_TILE, ROWS_PER_TILE)])
    plsc.subcore_barrier()

    pltpu.sync_copy(dst_hbm.at[pl.ds(w * CPW, CPW)], dstb)
    pltpu.sync_copy(ew_hbm.at[pl.ds(w * CPW, CPW)], ewb)

    def chunk_body(c, _):
        def fill(g, _):
            wv = ewb[c, pl.ds(16 * g, 16)]
            for l in range(16):
                vbuf[16 * g + l, :] = jnp.full((16,), wv[l], jnp.float32)
            return 0
        lax.fori_loop(0, CHUNK // 16, fill, 0)
        pltpu.sync_copy(vbuf, acc.at[dstb.at[c]], add=True)
        return 0

    lax.fori_loop(0, CPW, chunk_body, 0)
    plsc.subcore_barrier()

    pltpu.sync_copy(acc.at[pl.ds(sid * ROWS_PER_TILE, ROWS_PER_TILE)], zbuf)
    pltpu.sync_copy(zbuf, out_hbm.at[cid, pl.ds(sid * ROWS_PER_TILE, ROWS_PER_TILE)])


_deg_call = pl.kernel(
    _deg_body,
    out_type=jax.ShapeDtypeStruct((NC, NP, 16), jnp.float32),
    mesh=_mesh,
    scratch_types=[
        pltpu.VMEM((CPW, CHUNK), jnp.int32),
        pltpu.VMEM((CPW, CHUNK), jnp.float32),
        pltpu.VMEM((CHUNK, 16), jnp.float32),
        pltpu.VMEM((ROWS_PER_TILE, 16), jnp.float32),
        pltpu.VMEM_SHARED((NP, 16), jnp.float32),
        pltpu.SemaphoreType.DMA,
    ],
    compiler_params=_sc_params,
)


def _dinv_tc(deg_ref, o_ref):
    d = deg_ref[0] + deg_ref[1]
    o_ref[...] = lax.rsqrt(d)


def _dinv_call(degacc):
    return pl.pallas_call(
        _dinv_tc,
        out_shape=jax.ShapeDtypeStruct((NP, 16), jnp.float32),
    )(degacc)


def _norm_body(src_hbm, dst_hbm, ew_hbm, dinv_hbm, out_hbm,
               srcb, dstb, ewb, normb, dinvb, sem):
    cid = lax.axis_index("c")
    sid = lax.axis_index("s")
    w = sid * NC + cid

    pltpu.sync_copy(dinv_hbm, dinvb)
    pltpu.sync_copy(src_hbm.at[pl.ds(w * CPW, CPW)], srcb)
    pltpu.sync_copy(dst_hbm.at[pl.ds(w * CPW, CPW)], dstb)
    pltpu.sync_copy(ew_hbm.at[pl.ds(w * CPW, CPW)], ewb)

    def chunk_body(c, _):
        for g in range(CHUNK // 16):
            sl = pl.ds(16 * g, 16)
            isrc = srcb[c, sl]
            idst = dstb[c, sl]
            ds_ = plsc.load_gather(dinvb, [isrc])
            dd = plsc.load_gather(dinvb, [idst])
            normb[c, sl] = ds_ * ewb[c, sl] * dd
        return 0

    lax.fori_loop(0, CPW, chunk_body, 0)
    pltpu.sync_copy(normb, out_hbm.at[pl.ds(w * CPW, CPW)])


_norm_call = pl.kernel(
    _norm_body,
    out_type=jax.ShapeDtypeStruct((NCHUNKS, CHUNK), jnp.float32),
    mesh=_mesh,
    scratch_types=[
        pltpu.VMEM((CPW, CHUNK), jnp.int32),
        pltpu.VMEM((CPW, CHUNK), jnp.int32),
        pltpu.VMEM((CPW, CHUNK), jnp.float32),
        pltpu.VMEM((CPW, CHUNK), jnp.float32),
        pltpu.VMEM((NP,), jnp.float32),
        pltpu.SemaphoreType.DMA,
    ],
    compiler_params=_sc_params,
)


SPH = 24


def _make_agg(col_split, cpt, dup_rows=False):
    D = 128
    phases = cpt // SPH

    def body(src_hbm, dst_hbm, norm_hbm, rows_hbm, out_hbm,
             srcb, dstb, normb, rbf0, rbf1, rows_f, acc, semg0, semg1):
        cid = lax.axis_index("c")
        sid = lax.axis_index("s")
        if col_split:
            tbase = sid * cpt
        else:
            tbase = (sid * NC + cid) * cpt
        rbf = (rbf0, rbf1)
        semg = (semg0, semg1)

        _zero_rows(rows_f, CHUNK, D)
        for r in range(ROWS_PER_TILE // CHUNK):
            pltpu.sync_copy(
                rows_f, acc.at[pl.ds(sid * ROWS_PER_TILE + r * CHUNK, CHUNK)])
        plsc.subcore_barrier()

        def scale_rows(buf, c):
            def scale(g, _):
                nv = normb[c, pl.ds(16 * g, 16)]
                for l in range(16):
                    sv = jnp.take(nv, jnp.full((16,), l, jnp.int32))
                    e = 16 * g + l
                    for j in range(4):
                        v = buf[e, pl.ds(32 * j, 32)]
                        pa, pb = plsc.unpack(
                            v, format=plsc.PackFormat.INTERLEAVED)
                        rows_f[e, pl.ds(16 * j, 16)] = pa * sv
                        rows_f[e, pl.ds(64 + 16 * j, 16)] = pb * sv
                return 0
            lax.fori_loop(0, CHUNK // 16, scale, 0)

        for ph in range(phases):
            base = tbase + ph * SPH
            pltpu.sync_copy(src_hbm.at[pl.ds(base, SPH)], srcb)
            pltpu.sync_copy(dst_hbm.at[pl.ds(base, SPH)], dstb)
            pltpu.sync_copy(norm_hbm.at[pl.ds(base, SPH)], normb)

            if col_split or dup_rows:
                off = (cid * NP).astype(jnp.int32)

                def adjust(c, _):
                    for g in range(CHUNK // 16):
                        sl = pl.ds(16 * g, 16)
                        srcb[c, sl] = srcb[c, sl] + off
                    return 0
                lax.fori_loop(0, SPH, adjust, 0)

            pltpu.async_copy(rows_hbm.at[srcb.at[0]], rbf0, semg0)

            def pair_body(k, _):
                for b in range(2):
                    c = 2 * k + b
                    nb = 1 - b

                    @pl.when(c + 1 < SPH)
                    def _():
                        pltpu.async_copy(
                            rows_hbm.at[srcb.at[c + 1]], rbf[nb], semg[nb])

                    pltpu.make_async_copy(
                        rows_hbm.at[srcb.at[c]], rbf[b], semg[b]).wait()
                    scale_rows(rbf[b], c)
                    pltpu.sync_copy(rows_f, acc.at[dstb.at[c]], add=True)
                return 0

            lax.fori_loop(0, SPH // 2, pair_body, 0)

        plsc.subcore_barrier()

        for r in range(ROWS_PER_TILE // CHUNK):
            rbase = sid * ROWS_PER_TILE + r * CHUNK
            pltpu.sync_copy(acc.at[pl.ds(rbase, CHUNK)], rows_f)
            pltpu.sync_copy(rows_f, out_hbm.at[cid, pl.ds(rbase, CHUNK)])

    return pl.kernel(
        body,
        out_type=jax.ShapeDtypeStruct((NC, NP, D), jnp.float32),
        mesh=_mesh,
        scratch_types=[
            pltpu.VMEM((SPH, CHUNK), jnp.int32),
            pltpu.VMEM((SPH, CHUNK), jnp.int32),
            pltpu.VMEM((SPH, CHUNK), jnp.float32),
            pltpu.VMEM((CHUNK, D), jnp.bfloat16),
            pltpu.VMEM((CHUNK, D), jnp.bfloat16),
            pltpu.VMEM((CHUNK, D), jnp.float32),
            pltpu.VMEM_SHARED((NP, D), jnp.float32),
            pltpu.SemaphoreType.DMA,
            pltpu.SemaphoreType.DMA,
        ],
        compiler_params=_sc_params,
    )


_agg_l1 = _make_agg(col_split=True, cpt=CPT)
_agg_l2 = _make_agg(col_split=False, cpt=CPW, dup_rows=True)


_RB = 1024


def _mm1_tc(x_ref, w_ref, o_ref):
    o_ref[0] = jnp.dot(x_ref[...], w_ref[...],
                       preferred_element_type=jnp.float32
                       ).astype(jnp.bfloat16)


def _mm1_call(x, W1):
    return pl.pallas_call(
        _mm1_tc,
        out_shape=jax.ShapeDtypeStruct((NC, NP, 128), jnp.bfloat16),
        grid=(NP // _RB, NC),
        in_specs=[
            pl.BlockSpec((_RB, F), lambda i, j: (i, 0)),
            pl.BlockSpec((F, 128), lambda i, j: (0, j)),
        ],
        out_specs=pl.BlockSpec((1, _RB, 128), lambda i, j: (j, i, 0)),
    )(x, W1)


def _mm2_tc(a_ref, b1_ref, w2a_ref, w2b_ref, o_ref):
    hlo = jax.nn.relu(a_ref[0] + b1_ref[0:1, :])
    hhi = jax.nn.relu(a_ref[1] + b1_ref[1:2, :])
    o_ref[0] = (
        jnp.dot(hlo, w2a_ref[...], preferred_element_type=jnp.float32)
        + jnp.dot(hhi, w2b_ref[...], preferred_element_type=jnp.float32)
    ).astype(jnp.bfloat16)


def _mm2_call(agg1, b1r, W2a, W2b):
    return pl.pallas_call(
        _mm2_tc,
        out_shape=jax.ShapeDtypeStruct((NC, NP, O), jnp.bfloat16),
        grid=(NP // _RB, NC),
        in_specs=[
            pl.BlockSpec((2, _RB, 128), lambda i, j: (0, i, 0)),
            pl.BlockSpec((2, 128), lambda i, j: (0, 0)),
            pl.BlockSpec((128, O), lambda i, j: (0, 0)),
            pl.BlockSpec((128, O), lambda i, j: (0, 0)),
        ],
        out_specs=pl.BlockSpec((1, _RB, O), lambda i, j: (j, i, 0)),
    )(agg1, b1r, W2a, W2b)


def _final_tc(p_ref, b2_ref, o_ref):
    o_ref[...] = jax.nn.sigmoid(p_ref[0] + p_ref[1] + b2_ref[...])


def _final_call(parts, b2r):
    return pl.pallas_call(
        _final_tc,
        out_shape=jax.ShapeDtypeStruct((NP, O), jnp.float32),
        grid=(NP // _RB,),
        in_specs=[
            pl.BlockSpec((2, _RB, O), lambda i: (0, i, 0)),
            pl.BlockSpec((1, O), lambda i: (0, 0)),
        ],
        out_specs=pl.BlockSpec((_RB, O), lambda i: (i, 0)),
    )(parts, b2r)


def kernel(x, ei, ew, W1, b1, W2, b2):
    src = ei[0].astype(jnp.int32)
    dst = ei[1].astype(jnp.int32)
    loop = jnp.arange(N, dtype=jnp.int32)
    pad = E_PAD - (E + N)
    zi = jnp.zeros((pad,), jnp.int32)
    src_f = jnp.concatenate([src, loop, zi]).reshape(NCHUNKS, CHUNK)
    dst_f = jnp.concatenate([dst, loop, zi]).reshape(NCHUNKS, CHUNK)
    ew_f = jnp.concatenate(
        [ew, jnp.ones((N,), jnp.float32), jnp.zeros((pad,), jnp.float32)]
    ).reshape(NCHUNKS, CHUNK)

    degacc = _deg_call(dst_f, ew_f)
    dinv16 = _dinv_call(degacc)
    dinv = dinv16[:, 0]
    norm = _norm_call(src_f, dst_f, ew_f, dinv)

    xp = jnp.pad(x, ((0, NP - N), (0, 0)))
    pA = jnp.arange(64).reshape(4, 16)
    p128 = jnp.stack([pA, pA + 64], axis=-1).reshape(-1)
    W1p = jnp.concatenate([W1[:, :128][:, p128], W1[:, 128:][:, p128]], axis=1)
    xw2 = _mm1_call(xp, W1p).reshape(NC * NP, 128)
    agg1 = _agg_l1(src_f, dst_f, norm, xw2)

    b1r = b1.reshape(2, 128)
    W2a = W2[:128][:, p128]
    W2b = W2[128:][:, p128]
    hw2 = _mm2_call(agg1, b1r, W2a, W2b).reshape(NC * NP, O)

    parts = _agg_l2(src_f, dst_f, norm, hw2)
    out = _final_call(parts, b2.reshape(1, O))
    return out[:N]


# speedup vs baseline: 1.7613x; 1.0002x over previous
import functools

import jax
import jax.numpy as jnp
from jax import lax
from jax.experimental import pallas as pl
from jax.experimental.pallas import tpu as pltpu
from jax.experimental.pallas import tpu_sc as plsc

N = 10000
E = 160000
F = 256
H = 256
O = 128

NC = 2
NS = 16
NW = NC * NS
CHUNK = 128
E_PAD = 196608
NCHUNKS = E_PAD // CHUNK
CPW = NCHUNKS // NW
CPT = NCHUNKS // NS
NP = 10240
ROWS_PER_TILE = NP // NS

_mesh = plsc.VectorSubcoreMesh(core_axis_name="c", subcore_axis_name="s")
_sc_params = pltpu.CompilerParams(needs_layout_passes=False,
                                  use_tc_tiling_on_sc=False)


def _zero_rows(buf, nrows, width):
    def body(i, _):
        for j in range(width // 16):
            buf[i, pl.ds(16 * j, 16)] = jnp.zeros((16,), jnp.float32)
        return 0
    lax.fori_loop(0, nrows, body, 0)


def _deg_body(dst_hbm, ew_hbm, out_hbm, dstb, ewb, vbuf, zbuf, acc, sem):
    cid = lax.axis_index("c")
    sid = lax.axis_index("s")
    w = sid * NC + cid

    _zero_rows(zbuf, ROWS_PER_TILE, 16)
    pltpu.sync_copy(zbuf, acc.at[pl.ds(sid * ROWS_PER_TILE, ROWS_PER_TILE)])
    plsc.subcore_barrier()

    pltpu.sync_copy(dst_hbm.at[pl.ds(w * CPW, CPW)], dstb)
    pltpu.sync_copy(ew_hbm.at[pl.ds(w * CPW, CPW)], ewb)

    def chunk_body(c, _):
        def fill(g, _):
            wv = ewb[c, pl.ds(16 * g, 16)]
            for l in range(16):
                vbuf[16 * g + l, :] = jnp.full((16,), wv[l], jnp.float32)
            return 0
        lax.fori_loop(0, CHUNK // 16, fill, 0)
        pltpu.sync_copy(vbuf, acc.at[dstb.at[c]], add=True)
        return 0

    lax.fori_loop(0, CPW, chunk_body, 0)
    plsc.subcore_barrier()

    pltpu.sync_copy(acc.at[pl.ds(sid * ROWS_PER_TILE, ROWS_PER_TILE)], zbuf)
    pltpu.sync_copy(zbuf, out_hbm.at[cid, pl.ds(sid * ROWS_PER_TILE, ROWS_PER_TILE)])


_deg_call = pl.kernel(
    _deg_body,
    out_type=jax.ShapeDtypeStruct((NC, NP, 16), jnp.float32),
    mesh=_mesh,
    scratch_types=[
        pltpu.VMEM((CPW, CHUNK), jnp.int32),
        pltpu.VMEM((CPW, CHUNK), jnp.float32),
        pltpu.VMEM((CHUNK, 16), jnp.float32),
        pltpu.VMEM((ROWS_PER_TILE, 16), jnp.float32),
        pltpu.VMEM_SHARED((NP, 16), jnp.float32),
        pltpu.SemaphoreType.DMA,
    ],
    compiler_params=_sc_params,
)


def _dinv_tc(deg_ref, o_ref):
    d = deg_ref[0] + deg_ref[1]
    o_ref[...] = lax.rsqrt(d)


def _dinv_call(degacc):
    return pl.pallas_call(
        _dinv_tc,
        out_shape=jax.ShapeDtypeStruct((NP, 16), jnp.float32),
    )(degacc)


def _norm_body(src_hbm, dst_hbm, ew_hbm, dinv_hbm, out_hbm,
               srcb, dstb, ewb, normb, dinvb, sem):
    cid = lax.axis_index("c")
    sid = lax.axis_index("s")
    w = sid * NC + cid

    pltpu.sync_copy(dinv_hbm, dinvb)
    pltpu.sync_copy(src_hbm.at[pl.ds(w * CPW, CPW)], srcb)
    pltpu.sync_copy(dst_hbm.at[pl.ds(w * CPW, CPW)], dstb)
    pltpu.sync_copy(ew_hbm.at[pl.ds(w * CPW, CPW)], ewb)

    def chunk_body(c, _):
        for g in range(CHUNK // 16):
            sl = pl.ds(16 * g, 16)
            isrc = srcb[c, sl]
            idst = dstb[c, sl]
            ds_ = plsc.load_gather(dinvb, [isrc])
            dd = plsc.load_gather(dinvb, [idst])
            normb[c, sl] = ds_ * ewb[c, sl] * dd
        return 0

    lax.fori_loop(0, CPW, chunk_body, 0)
    pltpu.sync_copy(normb, out_hbm.at[pl.ds(w * CPW, CPW)])


_norm_call = pl.kernel(
    _norm_body,
    out_type=jax.ShapeDtypeStruct((NCHUNKS, CHUNK), jnp.float32),
    mesh=_mesh,
    scratch_types=[
        pltpu.VMEM((CPW, CHUNK), jnp.int32),
        pltpu.VMEM((CPW, CHUNK), jnp.int32),
        pltpu.VMEM((CPW, CHUNK), jnp.float32),
        pltpu.VMEM((CPW, CHUNK), jnp.float32),
        pltpu.VMEM((NP,), jnp.float32),
        pltpu.SemaphoreType.DMA,
    ],
    compiler_params=_sc_params,
)


SPH = 24


def _make_agg(col_split, cpt, dup_rows=False):
    D = 128
    phases = cpt // SPH

    def body(src_hbm, dst_hbm, norm_hbm, rows_hbm, out_hbm,
             srcb, dstb, normb, rbf0, rbf1, rows_f, acc, semg0, semg1):
        cid = lax.axis_index("c")
        sid = lax.axis_index("s")
        if col_split:
            tbase = sid * cpt
        else:
            tbase = (sid * NC + cid) * cpt
        rbf = (rbf0, rbf1)
        semg = (semg0, semg1)

        _zero_rows(rows_f, CHUNK, D)
        for r in range(ROWS_PER_TILE // CHUNK):
            pltpu.sync_copy(
                rows_f, acc.at[pl.ds(sid * ROWS_PER_TILE + r * CHUNK, CHUNK)])
        plsc.subcore_barrier()

        def scale_rows(buf, c):
            def scale(g, _):
                nv = normb[c, pl.ds(16 * g, 16)]
                for l in range(16):
                    sv = jnp.take(nv, jnp.full((16,), l, jnp.int32))
                    e = 16 * g + l
                    for j in range(4):
                        v = buf[e, pl.ds(32 * j, 32)]
                        pa, pb = plsc.unpack(
                            v, format=plsc.PackFormat.INTERLEAVED)
                        rows_f[e, pl.ds(16 * j, 16)] = pa * sv
                        rows_f[e, pl.ds(64 + 16 * j, 16)] = pb * sv
                return 0
            lax.fori_loop(0, CHUNK // 16, scale, 0)

        for ph in range(phases):
            base = tbase + ph * SPH
            pltpu.sync_copy(src_hbm.at[pl.ds(base, SPH)], srcb)
            pltpu.sync_copy(dst_hbm.at[pl.ds(base, SPH)], dstb)
            pltpu.sync_copy(norm_hbm.at[pl.ds(base, SPH)], normb)

            if col_split or dup_rows:
                off = (cid * NP).astype(jnp.int32)

                def adjust(c, _):
                    for g in range(CHUNK // 16):
                        sl = pl.ds(16 * g, 16)
                        srcb[c, sl] = srcb[c, sl] + off
                    return 0
                lax.fori_loop(0, SPH, adjust, 0)

            def gat(c, buf, sem):
                pltpu.async_copy(
                    rows_hbm.at[srcb.at[c, pl.ds(0, 64)]],
                    buf.at[pl.ds(0, 64)], sem)
                pltpu.async_copy(
                    rows_hbm.at[srcb.at[c, pl.ds(64, 64)]],
                    buf.at[pl.ds(64, 64)], sem)

            def gwait(c, buf, sem):
                pltpu.make_async_copy(
                    rows_hbm.at[srcb.at[c]], buf, sem).wait()

            gat(0, rbf0, semg0)

            def pair_body(k, _):
                for b in range(2):
                    c = 2 * k + b
                    nb = 1 - b

                    @pl.when(c + 1 < SPH)
                    def _():
                        gat(c + 1, rbf[nb], semg[nb])

                    gwait(c, rbf[b], semg[b])
                    scale_rows(rbf[b], c)
                    pltpu.sync_copy(rows_f, acc.at[dstb.at[c]], add=True)
                return 0

            lax.fori_loop(0, SPH // 2, pair_body, 0)

        plsc.subcore_barrier()

        for r in range(ROWS_PER_TILE // CHUNK):
            rbase = sid * ROWS_PER_TILE + r * CHUNK
            pltpu.sync_copy(acc.at[pl.ds(rbase, CHUNK)], rows_f)
            pltpu.sync_copy(rows_f, out_hbm.at[cid, pl.ds(rbase, CHUNK)])

    return pl.kernel(
        body,
        out_type=jax.ShapeDtypeStruct((NC, NP, D), jnp.float32),
        mesh=_mesh,
        scratch_types=[
            pltpu.VMEM((SPH, CHUNK), jnp.int32),
            pltpu.VMEM((SPH, CHUNK), jnp.int32),
            pltpu.VMEM((SPH, CHUNK), jnp.float32),
            pltpu.VMEM((CHUNK, D), jnp.bfloat16),
            pltpu.VMEM((CHUNK, D), jnp.bfloat16),
            pltpu.VMEM((CHUNK, D), jnp.float32),
            pltpu.VMEM_SHARED((NP, D), jnp.float32),
            pltpu.SemaphoreType.DMA,
            pltpu.SemaphoreType.DMA,
        ],
        compiler_params=_sc_params,
    )


_agg_l1 = _make_agg(col_split=True, cpt=CPT)
_agg_l2 = _make_agg(col_split=False, cpt=CPW, dup_rows=True)


_RB = 1024


def _mm1_tc(x_ref, w_ref, o_ref):
    o_ref[0] = jnp.dot(x_ref[...], w_ref[...],
                       preferred_element_type=jnp.float32
                       ).astype(jnp.bfloat16)


def _mm1_call(x, W1):
    return pl.pallas_call(
        _mm1_tc,
        out_shape=jax.ShapeDtypeStruct((NC, NP, 128), jnp.bfloat16),
        grid=(NP // _RB, NC),
        in_specs=[
            pl.BlockSpec((_RB, F), lambda i, j: (i, 0)),
            pl.BlockSpec((F, 128), lambda i, j: (0, j)),
        ],
        out_specs=pl.BlockSpec((1, _RB, 128), lambda i, j: (j, i, 0)),
    )(x, W1)


def _mm2_tc(a_ref, b1_ref, w2a_ref, w2b_ref, o_ref):
    hlo = jax.nn.relu(a_ref[0] + b1_ref[0:1, :])
    hhi = jax.nn.relu(a_ref[1] + b1_ref[1:2, :])
    o_ref[0] = (
        jnp.dot(hlo, w2a_ref[...], preferred_element_type=jnp.float32)
        + jnp.dot(hhi, w2b_ref[...], preferred_element_type=jnp.float32)
    ).astype(jnp.bfloat16)


def _mm2_call(agg1, b1r, W2a, W2b):
    return pl.pallas_call(
        _mm2_tc,
        out_shape=jax.ShapeDtypeStruct((NC, NP, O), jnp.bfloat16),
        grid=(NP // _RB, NC),
        in_specs=[
            pl.BlockSpec((2, _RB, 128), lambda i, j: (0, i, 0)),
            pl.BlockSpec((2, 128), lambda i, j: (0, 0)),
            pl.BlockSpec((128, O), lambda i, j: (0, 0)),
            pl.BlockSpec((128, O), lambda i, j: (0, 0)),
        ],
        out_specs=pl.BlockSpec((1, _RB, O), lambda i, j: (j, i, 0)),
    )(agg1, b1r, W2a, W2b)


def _final_tc(p_ref, b2_ref, o_ref):
    o_ref[...] = jax.nn.sigmoid(p_ref[0] + p_ref[1] + b2_ref[...])


def _final_call(parts, b2r):
    return pl.pallas_call(
        _final_tc,
        out_shape=jax.ShapeDtypeStruct((NP, O), jnp.float32),
        grid=(NP // _RB,),
        in_specs=[
            pl.BlockSpec((2, _RB, O), lambda i: (0, i, 0)),
            pl.BlockSpec((1, O), lambda i: (0, 0)),
        ],
        out_specs=pl.BlockSpec((_RB, O), lambda i: (i, 0)),
    )(parts, b2r)


def kernel(x, ei, ew, W1, b1, W2, b2):
    src = ei[0].astype(jnp.int32)
    dst = ei[1].astype(jnp.int32)
    loop = jnp.arange(N, dtype=jnp.int32)
    pad = E_PAD - (E + N)
    zi = jnp.zeros((pad,), jnp.int32)
    src_f = jnp.concatenate([src, loop, zi]).reshape(NCHUNKS, CHUNK)
    dst_f = jnp.concatenate([dst, loop, zi]).reshape(NCHUNKS, CHUNK)
    ew_f = jnp.concatenate(
        [ew, jnp.ones((N,), jnp.float32), jnp.zeros((pad,), jnp.float32)]
    ).reshape(NCHUNKS, CHUNK)

    degacc = _deg_call(dst_f, ew_f)
    dinv16 = _dinv_call(degacc)
    dinv = dinv16[:, 0]
    norm = _norm_call(src_f, dst_f, ew_f, dinv)

    xp = jnp.pad(x, ((0, NP - N), (0, 0)))
    pA = jnp.arange(64).reshape(4, 16)
    p128 = jnp.stack([pA, pA + 64], axis=-1).reshape(-1)
    W1p = jnp.concatenate([W1[:, :128][:, p128], W1[:, 128:][:, p128]], axis=1)
    xw2 = _mm1_call(xp, W1p).reshape(NC * NP, 128)
    agg1 = _agg_l1(src_f, dst_f, norm, xw2)

    b1r = b1.reshape(2, 128)
    W2a = W2[:128][:, p128]
    W2b = W2[128:][:, p128]
    hw2 = _mm2_call(agg1, b1r, W2a, W2b).reshape(NC * NP, O)

    parts = _agg_l2(src_f, dst_f, norm, hw2)
    out = _final_call(parts, b2.reshape(1, O))
    return out[:N]
